# Initial kernel scaffold; baseline (speedup 1.0000x reference)
import jax
import jax.numpy as jnp
from jax.experimental import pallas as pl


def kernel(prop, idx_i, idx_j, basis, W, b):
    raise NotImplementedError("write your pallas kernel here")


# speedup vs baseline: 1.8242x; 1.8242x over previous
import functools

import jax
import jax.numpy as jnp
import numpy as np
from jax import lax
from jax.experimental import pallas as pl
from jax.experimental.pallas import tpu as pltpu
from jax.experimental.pallas import tpu_sc as plsc

N_NODES = 10000
N_EDGES = 320000
IN_FEAT = 128
OUT_FEAT = 64
N_BASIS = 4
FF = OUT_FEAT * N_BASIS

NC = 2
NS = 16
NW = NC * NS
LANES = 16

EPW = N_EDGES // NW
CH = 80
NCHUNK = EPW // CH
GRP = CH // 4

_k2 = np.arange(FF)
_PERM = 4 * (_k2 % OUT_FEAT) + (_k2 // OUT_FEAT)


def _node_tables(prop, wi, wj, bj):
    def mm(p_ref, wi_ref, wj_ref, b_ref, ti_ref, tj_ref):
        p = p_ref[...]
        ti_ref[...] = jnp.dot(p, wi_ref[...], preferred_element_type=jnp.float32)
        tj_ref[...] = (
            jnp.dot(p, wj_ref[...], preferred_element_type=jnp.float32)
            + b_ref[...]
        )

    rows = 2000
    grid = N_NODES // rows
    return pl.pallas_call(
        mm,
        grid=(grid,),
        in_specs=[
            pl.BlockSpec((rows, IN_FEAT), lambda i: (i, 0)),
            pl.BlockSpec((IN_FEAT, FF), lambda i: (0, 0)),
            pl.BlockSpec((IN_FEAT, FF), lambda i: (0, 0)),
            pl.BlockSpec((1, FF), lambda i: (0, 0)),
        ],
        out_specs=[
            pl.BlockSpec((rows, FF), lambda i: (i, 0)),
            pl.BlockSpec((rows, FF), lambda i: (i, 0)),
        ],
        out_shape=[
            jax.ShapeDtypeStruct((N_NODES, FF), jnp.float32),
            jax.ShapeDtypeStruct((N_NODES, FF), jnp.float32),
        ],
    )(prop, wi, wj, bj)


_SC_MESH = plsc.VectorSubcoreMesh(
    core_axis_name="c", subcore_axis_name="s", num_cores=NC, num_subcores=NS
)


@functools.partial(
    pl.kernel,
    out_type=jax.ShapeDtypeStruct((N_EDGES, OUT_FEAT), jnp.float32),
    mesh=_SC_MESH,
    scratch_types=[
        pltpu.VMEM((CH,), jnp.int32),
        pltpu.VMEM((CH,), jnp.int32),
        pltpu.VMEM((CH * N_BASIS,), jnp.float32),
        pltpu.VMEM((CH, FF), jnp.float32),
        pltpu.VMEM((CH, FF), jnp.float32),
        pltpu.VMEM((CH, OUT_FEAT), jnp.float32),
        pltpu.SemaphoreType.DMA,
        pltpu.SemaphoreType.DMA,
    ],
)
def _edge_kernel(ti_hbm, tj_hbm, ii_hbm, jj_hbm, bas_hbm, out_hbm,
                 ii_v, jj_v, bas_v, ri_v, rj_v, o_v, sem_i, sem_j):
    wid = lax.axis_index("s") * NC + lax.axis_index("c")
    base = wid * EPW

    def chunk_body(ci, carry):
        off = base + ci * CH
        pltpu.sync_copy(ii_hbm.at[pl.ds(off, CH)], ii_v)
        pltpu.sync_copy(jj_hbm.at[pl.ds(off, CH)], jj_v)
        pltpu.sync_copy(bas_hbm.at[pl.ds(off * N_BASIS, CH * N_BASIS)], bas_v)
        cp_i = pltpu.async_copy(ti_hbm.at[ii_v], ri_v, sem_i)
        cp_j = pltpu.async_copy(tj_hbm.at[jj_v], rj_v, sem_j)
        cp_i.wait()
        cp_j.wait()

        def grp_body(g, gcarry):
            bgrp = bas_v[pl.ds(g * LANES, LANES)]
            for eq in range(4):
                e = g * 4 + eq
                sp = [
                    jnp.take_along_axis(
                        bgrp,
                        jnp.full((LANES,), eq * N_BASIS + q, jnp.int32),
                        axis=0,
                        mode="promise_in_bounds",
                    )
                    for q in range(N_BASIS)
                ]
                for r in range(4):
                    acc = None
                    for q in range(N_BASIS):
                        col = OUT_FEAT * q + LANES * r
                        s = ri_v[e, pl.ds(col, LANES)] + rj_v[e, pl.ds(col, LANES)]
                        acc = sp[q] * s if acc is None else acc + sp[q] * s
                    o_v[e, pl.ds(LANES * r, LANES)] = acc
            return gcarry

        lax.fori_loop(0, GRP, grp_body, 0)
        pltpu.sync_copy(o_v, out_hbm.at[pl.ds(off, CH)])
        return carry

    lax.fori_loop(0, NCHUNK, chunk_body, 0)


def kernel(prop, idx_i, idx_j, basis, W, b):
    W = W.astype(jnp.float32)
    wp = W[:, _PERM]
    bp = b.astype(jnp.float32)[_PERM].reshape(1, FF)
    ti, tj = _node_tables(
        prop.astype(jnp.float32), wp[:IN_FEAT], wp[IN_FEAT:], bp
    )
    out = _edge_kernel(
        ti,
        tj,
        idx_i.astype(jnp.int32),
        idx_j.astype(jnp.int32),
        basis.astype(jnp.float32).reshape(-1),
    )
    return out


# speedup vs baseline: 2.9337x; 1.6083x over previous
import functools

import jax
import jax.numpy as jnp
import numpy as np
from jax import lax
from jax.experimental import pallas as pl
from jax.experimental.pallas import tpu as pltpu
from jax.experimental.pallas import tpu_sc as plsc

N_NODES = 10000
N_EDGES = 320000
IN_FEAT = 128
OUT_FEAT = 64
N_BASIS = 4
FF = OUT_FEAT * N_BASIS

NC = 2
NS = 16
NW = NC * NS
LANES = 16

EPW = N_EDGES // NW
CH = 40
NCHUNK = EPW // CH
GRP = CH // 4

_k2 = np.arange(FF)
_PERM = 4 * (_k2 % OUT_FEAT) + (_k2 // OUT_FEAT)


def _node_tables(prop, wi, wj, bj):
    def mm(p_ref, wi_ref, wj_ref, b_ref, ti_ref, tj_ref):
        p = p_ref[...]
        ti_ref[...] = jnp.dot(p, wi_ref[...], preferred_element_type=jnp.float32)
        tj_ref[...] = (
            jnp.dot(p, wj_ref[...], preferred_element_type=jnp.float32)
            + b_ref[...]
        )

    rows = 2000
    grid = N_NODES // rows
    return pl.pallas_call(
        mm,
        grid=(grid,),
        in_specs=[
            pl.BlockSpec((rows, IN_FEAT), lambda i: (i, 0)),
            pl.BlockSpec((IN_FEAT, FF), lambda i: (0, 0)),
            pl.BlockSpec((IN_FEAT, FF), lambda i: (0, 0)),
            pl.BlockSpec((1, FF), lambda i: (0, 0)),
        ],
        out_specs=[
            pl.BlockSpec((rows, FF), lambda i: (i, 0)),
            pl.BlockSpec((rows, FF), lambda i: (i, 0)),
        ],
        out_shape=[
            jax.ShapeDtypeStruct((N_NODES, FF), jnp.float32),
            jax.ShapeDtypeStruct((N_NODES, FF), jnp.float32),
        ],
    )(prop, wi, wj, bj)


_SC_MESH = plsc.VectorSubcoreMesh(
    core_axis_name="c", subcore_axis_name="s", num_cores=NC, num_subcores=NS
)


@functools.partial(
    pl.kernel,
    out_type=jax.ShapeDtypeStruct((N_EDGES, OUT_FEAT), jnp.float32),
    mesh=_SC_MESH,
    scratch_types=[
        pltpu.VMEM((EPW,), jnp.int32),
        pltpu.VMEM((EPW,), jnp.int32),
        pltpu.VMEM((EPW * N_BASIS,), jnp.float32),
        pltpu.VMEM((CH, FF), jnp.float32),
        pltpu.VMEM((CH, FF), jnp.float32),
        pltpu.VMEM((CH, FF), jnp.float32),
        pltpu.VMEM((CH, FF), jnp.float32),
        pltpu.VMEM((CH, OUT_FEAT), jnp.float32),
        pltpu.VMEM((CH, OUT_FEAT), jnp.float32),
        pltpu.SemaphoreType.DMA,
        pltpu.SemaphoreType.DMA,
        pltpu.SemaphoreType.DMA,
        pltpu.SemaphoreType.DMA,
    ],
)
def _edge_kernel(ti_hbm, tj_hbm, ii_hbm, jj_hbm, bas_hbm, out_hbm,
                 ii_v, jj_v, bas_v, ri0, ri1, rj0, rj1, o0, o1,
                 sg0, sg1, so0, so1):
    wid = lax.axis_index("s") * NC + lax.axis_index("c")
    base = wid * EPW

    ri = (ri0, ri1)
    rj = (rj0, rj1)
    ov = (o0, o1)
    sg = (sg0, sg1)
    so = (so0, so1)

    pltpu.sync_copy(ii_hbm.at[pl.ds(base, EPW)], ii_v)
    pltpu.sync_copy(jj_hbm.at[pl.ds(base, EPW)], jj_v)
    pltpu.sync_copy(bas_hbm.at[pl.ds(base * N_BASIS, EPW * N_BASIS)], bas_v)

    def fire_gathers(chunk, slot):
        idx_i = ii_v.at[pl.ds(chunk * CH, CH)]
        idx_j = jj_v.at[pl.ds(chunk * CH, CH)]
        pltpu.async_copy(ti_hbm.at[idx_i], ri[slot], sg[slot])
        pltpu.async_copy(tj_hbm.at[idx_j], rj[slot], sg[slot])

    def wait_gathers(chunk, slot):
        idx_i = ii_v.at[pl.ds(chunk * CH, CH)]
        idx_j = jj_v.at[pl.ds(chunk * CH, CH)]
        pltpu.make_async_copy(ti_hbm.at[idx_i], ri[slot], sg[slot]).wait()
        pltpu.make_async_copy(tj_hbm.at[idx_j], rj[slot], sg[slot]).wait()

    def out_desc(chunk, slot):
        return pltpu.make_async_copy(
            ov[slot], out_hbm.at[pl.ds(base + chunk * CH, CH)], so[slot]
        )

    def compute(chunk, slot):
        riv, rjv, o = ri[slot], rj[slot], ov[slot]

        def grp_body(g, carry):
            bgrp = bas_v[pl.ds(chunk * (CH * N_BASIS) + g * LANES, LANES)]
            for eq in range(4):
                e = g * 4 + eq
                sp = [
                    jnp.take_along_axis(
                        bgrp,
                        jnp.full((LANES,), eq * N_BASIS + q, jnp.int32),
                        axis=0,
                        mode="promise_in_bounds",
                    )
                    for q in range(N_BASIS)
                ]
                for r in range(4):
                    acc = None
                    for q in range(N_BASIS):
                        col = OUT_FEAT * q + LANES * r
                        s = riv[e, pl.ds(col, LANES)] + rjv[e, pl.ds(col, LANES)]
                        acc = sp[q] * s if acc is None else acc + sp[q] * s
                    o[e, pl.ds(LANES * r, LANES)] = acc
            return carry

        lax.fori_loop(0, GRP, grp_body, 0)

    fire_gathers(0, 0)

    def pair_body(ci2, carry):
        for b in range(2):
            chunk = ci2 * 2 + b

            @pl.when(chunk + 1 < NCHUNK)
            def _():
                fire_gathers(chunk + 1, 1 - b)

            wait_gathers(chunk, b)

            @pl.when(chunk >= 2)
            def _():
                out_desc(chunk - 2, b).wait()

            compute(chunk, b)
            out_desc(chunk, b).start()
        return carry

    lax.fori_loop(0, NCHUNK // 2, pair_body, 0)
    out_desc(NCHUNK - 2, 0).wait()
    out_desc(NCHUNK - 1, 1).wait()


def kernel(prop, idx_i, idx_j, basis, W, b):
    W = W.astype(jnp.float32)
    wp = W[:, _PERM]
    bp = b.astype(jnp.float32)[_PERM].reshape(1, FF)
    ti, tj = _node_tables(
        prop.astype(jnp.float32), wp[:IN_FEAT], wp[IN_FEAT:], bp
    )
    out = _edge_kernel(
        ti,
        tj,
        idx_i.astype(jnp.int32),
        idx_j.astype(jnp.int32),
        basis.astype(jnp.float32).reshape(-1),
    )
    return out


# speedup vs baseline: 2.9499x; 1.0055x over previous
import functools

import jax
import jax.numpy as jnp
import numpy as np
from jax import lax
from jax.experimental import pallas as pl
from jax.experimental.pallas import tpu as pltpu
from jax.experimental.pallas import tpu_sc as plsc

N_NODES = 10000
N_EDGES = 320000
IN_FEAT = 128
OUT_FEAT = 64
N_BASIS = 4
FF = OUT_FEAT * N_BASIS

NC = 2
NS = 16
NW = NC * NS
LANES = 16

EPW = N_EDGES // NW
CH = 80
NCHUNK = EPW // CH
GRP = CH // 4

_k2 = np.arange(FF)
_PERM = 4 * (_k2 % OUT_FEAT) + (_k2 // OUT_FEAT)


def _node_tables(prop, wi, wj, bj):
    def mm(p_ref, wi_ref, wj_ref, b_ref, ti_ref, tj_ref):
        p = p_ref[...]
        ti_ref[...] = jnp.dot(p, wi_ref[...], preferred_element_type=jnp.float32)
        tj_ref[...] = (
            jnp.dot(p, wj_ref[...], preferred_element_type=jnp.float32)
            + b_ref[...]
        )

    rows = 2000
    grid = N_NODES // rows
    return pl.pallas_call(
        mm,
        grid=(grid,),
        in_specs=[
            pl.BlockSpec((rows, IN_FEAT), lambda i: (i, 0)),
            pl.BlockSpec((IN_FEAT, FF), lambda i: (0, 0)),
            pl.BlockSpec((IN_FEAT, FF), lambda i: (0, 0)),
            pl.BlockSpec((1, FF), lambda i: (0, 0)),
        ],
        out_specs=[
            pl.BlockSpec((rows, FF), lambda i: (i, 0)),
            pl.BlockSpec((rows, FF), lambda i: (i, 0)),
        ],
        out_shape=[
            jax.ShapeDtypeStruct((N_NODES, FF), jnp.float32),
            jax.ShapeDtypeStruct((N_NODES, FF), jnp.float32),
        ],
    )(prop, wi, wj, bj)


_SC_MESH = plsc.VectorSubcoreMesh(
    core_axis_name="c", subcore_axis_name="s", num_cores=NC, num_subcores=NS
)


@functools.partial(
    pl.kernel,
    out_type=jax.ShapeDtypeStruct((N_EDGES, OUT_FEAT), jnp.float32),
    mesh=_SC_MESH,
    scratch_types=[
        pltpu.VMEM((EPW,), jnp.int32),
        pltpu.VMEM((EPW,), jnp.int32),
        pltpu.VMEM((CH * N_BASIS,), jnp.float32),
        pltpu.VMEM((CH * N_BASIS,), jnp.float32),
        pltpu.VMEM((CH, FF), jnp.float32),
        pltpu.VMEM((CH, FF), jnp.float32),
        pltpu.VMEM((CH, FF), jnp.float32),
        pltpu.VMEM((CH, FF), jnp.float32),
        pltpu.VMEM((CH, OUT_FEAT), jnp.float32),
        pltpu.VMEM((CH, OUT_FEAT), jnp.float32),
        pltpu.SemaphoreType.DMA,
        pltpu.SemaphoreType.DMA,
        pltpu.SemaphoreType.DMA,
        pltpu.SemaphoreType.DMA,
    ],
)
def _edge_kernel(ti_hbm, tj_hbm, ii_hbm, jj_hbm, bas_hbm, out_hbm,
                 ii_v, jj_v, ba0, ba1, ri0, ri1, rj0, rj1, o0, o1,
                 sg0, sg1, so0, so1):
    wid = lax.axis_index("s") * NC + lax.axis_index("c")
    base = wid * EPW

    bas = (ba0, ba1)
    ri = (ri0, ri1)
    rj = (rj0, rj1)
    ov = (o0, o1)
    sg = (sg0, sg1)
    so = (so0, so1)

    pltpu.sync_copy(ii_hbm.at[pl.ds(base, EPW)], ii_v)
    pltpu.sync_copy(jj_hbm.at[pl.ds(base, EPW)], jj_v)

    def descs(chunk, slot):
        idx_i = ii_v.at[pl.ds(chunk * CH, CH)]
        idx_j = jj_v.at[pl.ds(chunk * CH, CH)]
        boff = (base + chunk * CH) * N_BASIS
        return (
            pltpu.make_async_copy(ti_hbm.at[idx_i], ri[slot], sg[slot]),
            pltpu.make_async_copy(tj_hbm.at[idx_j], rj[slot], sg[slot]),
            pltpu.make_async_copy(
                bas_hbm.at[pl.ds(boff, CH * N_BASIS)], bas[slot], sg[slot]
            ),
        )

    def fire_gathers(chunk, slot):
        for d in descs(chunk, slot):
            d.start()

    def wait_gathers(chunk, slot):
        for d in descs(chunk, slot):
            d.wait()

    def desc_o(chunk, slot):
        return pltpu.make_async_copy(
            ov[slot], out_hbm.at[pl.ds(base + chunk * CH, CH)], so[slot]
        )

    def compute(chunk, slot):
        riv, rjv, o = ri[slot], rj[slot], ov[slot]
        bv = bas[slot]

        def grp_body(g, carry):
            bgrp = bv[pl.ds(g * LANES, LANES)]
            for eq in range(4):
                e = g * 4 + eq
                sp = [
                    jnp.take_along_axis(
                        bgrp,
                        jnp.full((LANES,), eq * N_BASIS + q, jnp.int32),
                        axis=0,
                        mode="promise_in_bounds",
                    )
                    for q in range(N_BASIS)
                ]
                for r in range(4):
                    acc = None
                    for q in range(N_BASIS):
                        col = OUT_FEAT * q + LANES * r
                        s = riv[e, pl.ds(col, LANES)] + rjv[e, pl.ds(col, LANES)]
                        acc = sp[q] * s if acc is None else acc + sp[q] * s
                    o[e, pl.ds(LANES * r, LANES)] = acc
            return carry

        lax.fori_loop(0, GRP, grp_body, 0)

    def body(chunk, b):
        @pl.when(chunk + 1 < NCHUNK)
        def _():
            fire_gathers(chunk + 1, 1 - b)

        wait_gathers(chunk, b)

        @pl.when(chunk >= 2)
        def _():
            desc_o(chunk - 2, b).wait()

        compute(chunk, b)
        desc_o(chunk, b).start()

    fire_gathers(0, 0)

    def pair_body(ci2, carry):
        for b in range(2):
            body(ci2 * 2 + b, b)
        return carry

    lax.fori_loop(0, NCHUNK // 2, pair_body, 0)
    body(jnp.int32(NCHUNK - 1), 0)
    desc_o(NCHUNK - 2, 1).wait()
    desc_o(NCHUNK - 1, 0).wait()


def kernel(prop, idx_i, idx_j, basis, W, b):
    W = W.astype(jnp.float32)
    wp = W[:, _PERM]
    bp = b.astype(jnp.float32)[_PERM].reshape(1, FF)
    ti, tj = _node_tables(
        prop.astype(jnp.float32), wp[:IN_FEAT], wp[IN_FEAT:], bp
    )
    out = _edge_kernel(
        ti,
        tj,
        idx_i.astype(jnp.int32),
        idx_j.astype(jnp.int32),
        basis.astype(jnp.float32).reshape(-1),
    )
    return out
